# baseline (speedup 1.0000x reference)
import functools

import jax
import jax.numpy as jnp
from jax import lax
from jax.experimental import pallas as pl
from jax.experimental.pallas import tpu as pltpu
from jax.experimental.pallas import tpu_sc as plsc

NODE_DIM = 256
EDGE_DIM = 16
HIDDEN = 512
N_NODES = 10000
N_EDGES = 160000

_NC = 2
_NS = 16
_NW = _NC * _NS
_L = 16

_F = 128
_KP = HIDDEN // _F
_NPAD = 10240
_RPT = _NPAD // _NS
_EPAD = 163840
_EPT = _EPAD // _NW
_G = 32
_NG = _EPT // _G
_ZR = 16


def _sc_body(p0, p1, p2, p3, q0, q1, q2, q3, e0, e1, e2, e3, src_hbm, dst_hbm,
             s0_hbm, s1_hbm, d0_hbm, d1_hbm,
             srcb, dstb,
             bufp0, bufq0, bufe0, rbuf0, bufp1, bufq1, bufe1, rbuf1,
             onesb, zbuf, dzero, shared, shared_deg,
             semp0, semq0, seme0, sems0,
             semp1, semq1, seme1, sems1, semd0, semd1):
    cid = lax.axis_index("c")
    sid = lax.axis_index("s")
    wid = sid * _NC + cid
    ebase0 = wid * _EPT
    zero16 = jnp.zeros((_L,), jnp.float32)
    one16 = jnp.full((_L,), 1.0, jnp.float32)

    pltpu.sync_copy(src_hbm.at[pl.ds(ebase0, _EPT)], srcb)
    pltpu.sync_copy(dst_hbm.at[pl.ds(ebase0, _EPT)], dstb)

    def _zb(i, _):
        for c in range(_F // _L):
            zbuf[i, pl.ds(c * _L, _L)] = zero16
        return _
    lax.fori_loop(0, _ZR, _zb, None)

    def _dz(i, _):
        dzero[pl.ds(i * _L, _L)] = zero16
        return _
    lax.fori_loop(0, _RPT // _L, _dz, None)

    def _ob(i, _):
        onesb[pl.ds(i * _L, _L)] = one16
        return _
    lax.fori_loop(0, _G // _L, _ob, None)

    rlo = sid * _RPT

    bufs = ((bufp0, bufq0, bufe0, rbuf0, semp0, semq0, seme0, sems0, semd0),
            (bufp1, bufq1, bufe1, rbuf1, semp1, semq1, seme1, sems1, semd1))

    for k, (pk, qk, ek) in enumerate(((p0, q0, e0), (p1, q1, e1),
                                      (p2, q2, e2), (p3, q3, e3))):
        def _zs(i, _):
            pltpu.sync_copy(zbuf, shared.at[pl.ds(rlo + i * _ZR, _ZR)])
            return _
        lax.fori_loop(0, _RPT // _ZR, _zs, None)
        if k == 0:
            pltpu.sync_copy(dzero, shared_deg.at[pl.ds(rlo, _RPT)])
        plsc.subcore_barrier()

        def _fire_gather(g, par, pk=pk, qk=qk, ek=ek):
            bp, bq, be = bufs[par][0], bufs[par][1], bufs[par][2]
            sp, sq, se = bufs[par][4], bufs[par][5], bufs[par][6]
            gb = g * _G
            pltpu.async_copy(pk.at[srcb.at[pl.ds(gb, _G)]], bp, sp)
            pltpu.async_copy(qk.at[dstb.at[pl.ds(gb, _G)]], bq, sq)
            pltpu.async_copy(ek.at[pl.ds(ebase0 + gb, _G)], be, se)

        def _wait_gather(par, pk=pk, qk=qk, ek=ek):
            bp, bq, be = bufs[par][0], bufs[par][1], bufs[par][2]
            sp, sq, se = bufs[par][4], bufs[par][5], bufs[par][6]
            pltpu.make_async_copy(pk.at[srcb.at[pl.ds(0, _G)]], bp, sp).wait()
            pltpu.make_async_copy(qk.at[dstb.at[pl.ds(0, _G)]], bq, sq).wait()
            pltpu.make_async_copy(ek.at[pl.ds(0, _G)], be, se).wait()

        def _wait_scat(par, k=k):
            rb, ss, sd = bufs[par][3], bufs[par][7], bufs[par][8]
            pltpu.make_async_copy(
                rb, shared.at[dstb.at[pl.ds(0, _G)]], ss).wait()
            if k == 0:
                pltpu.make_async_copy(
                    onesb, shared_deg.at[dstb.at[pl.ds(0, _G)]], sd).wait()

        def _compute_fire(g, par, k=k):
            bp, bq, be, rb = (bufs[par][0], bufs[par][1], bufs[par][2],
                              bufs[par][3])
            ss, sd = bufs[par][7], bufs[par][8]
            gb = g * _G

            himask = jnp.full((_L,), 0xFFFF0000, jnp.uint32)

            def _f32lo(w):
                return lax.bitcast_convert_type(w << 16, jnp.float32)

            def _f32hi(w):
                return lax.bitcast_convert_type(w & himask, jnp.float32)

            def _row(j, _):
                for c in range(_F // (2 * _L)):
                    col = c * _L
                    pw = bp[j, pl.ds(col, _L)]
                    qw = bq[j, pl.ds(col, _L)]
                    ew = be[j, pl.ds(col, _L)]
                    lo = jnp.maximum(_f32lo(pw) + _f32lo(qw) + _f32lo(ew), 0.0)
                    hi = jnp.maximum(_f32hi(pw) + _f32hi(qw) + _f32hi(ew), 0.0)
                    rb[j, pl.ds(col, _L)] = lo
                    rb[j, pl.ds(_F // 2 + col, _L)] = hi
                return _
            lax.fori_loop(0, _G, _row, None)

            pltpu.async_copy(rb, shared.at[dstb.at[pl.ds(gb, _G)]], ss,
                             add=True)
            if k == 0:
                pltpu.async_copy(onesb,
                                 shared_deg.at[dstb.at[pl.ds(gb, _G)]], sd,
                                 add=True)

        _fire_gather(0, 0)

        def _pair(i, _):
            ge = 2 * i

            _fire_gather(ge + 1, 1)
            _wait_gather(0)

            @pl.when(i > 0)
            def _():
                _wait_scat(0)
            _compute_fire(ge, 0)

            _fire_gather(ge + 2, 0)
            _wait_gather(1)

            @pl.when(i > 0)
            def _():
                _wait_scat(1)
            _compute_fire(ge + 1, 1)
            return _
        lax.fori_loop(0, (_NG - 2) // 2, _pair, None)

        _fire_gather(_NG - 1, 1)
        _wait_gather(0)
        _wait_scat(0)
        _compute_fire(_NG - 2, 0)
        _wait_gather(1)
        _wait_scat(1)
        _compute_fire(_NG - 1, 1)
        _wait_scat(0)
        _wait_scat(1)
        plsc.subcore_barrier()

        @pl.when(cid == 0)
        def _(k=k):
            pltpu.sync_copy(shared.at[pl.ds(rlo, _RPT)],
                            s0_hbm.at[k, pl.ds(rlo, _RPT)])
            if k == 0:
                pltpu.sync_copy(shared_deg.at[pl.ds(rlo, _RPT)],
                                d0_hbm.at[pl.ds(rlo, _RPT)])

        @pl.when(cid == 1)
        def _(k=k):
            pltpu.sync_copy(shared.at[pl.ds(rlo, _RPT)],
                            s1_hbm.at[k, pl.ds(rlo, _RPT)])
            if k == 0:
                pltpu.sync_copy(shared_deg.at[pl.ds(rlo, _RPT)],
                                d1_hbm.at[pl.ds(rlo, _RPT)])
        plsc.subcore_barrier()


def _sc_segment(ps, qs, es, src, dst):
    mesh = plsc.VectorSubcoreMesh(core_axis_name="c", subcore_axis_name="s")
    f = pl.kernel(
        _sc_body,
        out_type=[
            jax.ShapeDtypeStruct((_KP, _NPAD, _F), jnp.float32),
            jax.ShapeDtypeStruct((_KP, _NPAD, _F), jnp.float32),
            jax.ShapeDtypeStruct((_NPAD,), jnp.float32),
            jax.ShapeDtypeStruct((_NPAD,), jnp.float32),
        ],
        mesh=mesh,
        compiler_params=pltpu.CompilerParams(use_tc_tiling_on_sc=False),
        scratch_types=(
            [pltpu.VMEM((_EPT,), jnp.int32)] * 2
            + [pltpu.VMEM((_G, _F // 2), jnp.uint32),
               pltpu.VMEM((_G, _F // 2), jnp.uint32),
               pltpu.VMEM((_G, _F // 2), jnp.uint32),
               pltpu.VMEM((_G, _F), jnp.float32),
               pltpu.VMEM((_G, _F // 2), jnp.uint32),
               pltpu.VMEM((_G, _F // 2), jnp.uint32),
               pltpu.VMEM((_G, _F // 2), jnp.uint32),
               pltpu.VMEM((_G, _F), jnp.float32)]
            + [
                pltpu.VMEM((_G,), jnp.float32),
                pltpu.VMEM((_ZR, _F), jnp.float32),
                pltpu.VMEM((_RPT,), jnp.float32),
                pltpu.VMEM_SHARED((_NPAD, _F), jnp.float32),
                pltpu.VMEM_SHARED((_NPAD,), jnp.float32),
            ]
            + [pltpu.SemaphoreType.DMA] * 10
        ),
    )
    return f(*ps, *qs, *es, src, dst)


def _bf16_bits(z):
    u = lax.bitcast_convert_type(z, jnp.uint32)
    return (u + jnp.uint32(0x7FFF) + ((u >> 16) & jnp.uint32(1))) >> 16


def _pack2(z):
    return _bf16_bits(z[:, :_F // 2]) | (_bf16_bits(z[:, _F // 2:]) << 16)


def _pq_body(x_ref, *refs):
    was = refs[0:4]
    wbs = refs[4:8]
    b1s = refs[8:12]
    pouts = refs[12:16]
    qouts = refs[16:20]
    xb = x_ref[:]
    for k in range(_KP):
        pouts[k][:] = _pack2(
            jnp.dot(xb, was[k][:], preferred_element_type=jnp.float32)
            + b1s[k][:])
        qouts[k][:] = _pack2(
            jnp.dot(xb, wbs[k][:], preferred_element_type=jnp.float32))


def _compute_pq(x, wa, wb, b1):
    blk = 2000
    wspec = pl.BlockSpec((NODE_DIM, _F), lambda i: (0, 0))
    bspec = pl.BlockSpec((1, _F), lambda i: (0, 0))
    ospec = pl.BlockSpec((blk, _F // 2), lambda i: (i, 0))
    was = [wa[:, k * _F:(k + 1) * _F] for k in range(_KP)]
    wbs = [wb[:, k * _F:(k + 1) * _F] for k in range(_KP)]
    b1s = [b1[:, k * _F:(k + 1) * _F] for k in range(_KP)]
    return pl.pallas_call(
        _pq_body,
        grid=(N_NODES // blk,),
        in_specs=[pl.BlockSpec((blk, NODE_DIM), lambda i: (i, 0))]
                 + [wspec] * 8 + [bspec] * 4,
        out_specs=[ospec] * 8,
        out_shape=[jax.ShapeDtypeStruct((_NPAD, _F // 2), jnp.uint32)] * 8,
    )(x, *was, *wbs, *b1s)


def _e_body(ea_ref, *refs):
    wcs = refs[0:4]
    eouts = refs[4:8]
    eb = ea_ref[:]
    for k in range(_KP):
        eouts[k][:] = _pack2(
            jnp.dot(eb, wcs[k][:], preferred_element_type=jnp.float32))


def _compute_e(edge_attr_pad, wc):
    blk = 8192
    return pl.pallas_call(
        _e_body,
        grid=(_EPAD // blk,),
        in_specs=[pl.BlockSpec((blk, EDGE_DIM), lambda i: (i, 0))]
                 + [pl.BlockSpec((EDGE_DIM, _F), lambda i: (0, 0))] * 4,
        out_specs=[pl.BlockSpec((blk, _F // 2), lambda i: (i, 0))] * 4,
        out_shape=[jax.ShapeDtypeStruct((_EPAD, _F // 2), jnp.uint32)] * 4,
    )(edge_attr_pad, *[wc[:, k * _F:(k + 1) * _F] for k in range(_KP)])


def _wb_body(w2_ref, b_ref, b2_ref, wb_ref, bb_ref):
    wb_ref[:] = jnp.dot(w2_ref[:], b_ref[:], preferred_element_type=jnp.float32)
    bb_ref[:] = jnp.dot(b2_ref[:], b_ref[:], preferred_element_type=jnp.float32)


def _compute_wb(fe_W2, B, fe_b2):
    return pl.pallas_call(
        _wb_body,
        out_shape=[
            jax.ShapeDtypeStruct((HIDDEN, HIDDEN), jnp.float32),
            jax.ShapeDtypeStruct((1, HIDDEN), jnp.float32),
        ],
    )(fe_W2, B, fe_b2.reshape(1, HIDDEN))


def _node_body(ph_ref, x_ref, s0_ref, s1_ref, d0_ref, d1_ref, a_ref, c_ref,
               wb_ref, bb_ref, b1_ref, w2_ref, b2_ref, o_ref):
    z = jnp.dot(ph_ref[:], a_ref[:], preferred_element_type=jnp.float32)
    z += jnp.dot(x_ref[:], c_ref[:], preferred_element_type=jnp.float32)
    for k in range(_KP):
        z += jnp.dot(s0_ref[k] + s1_ref[k], wb_ref[k * _F:(k + 1) * _F, :],
                     preferred_element_type=jnp.float32)
    z += (d0_ref[:] + d1_ref[:]) * bb_ref[:]
    z += b1_ref[:]
    z = jax.nn.relu(z)
    o_ref[:] = jnp.dot(z, w2_ref[:], preferred_element_type=jnp.float32) + b2_ref[:]


def _compute_node(pre_h, x, S0, S1, d0, d1, A, C, WB, bB, fn_b1, fn_W2, fn_b2):
    blk = 2000
    return pl.pallas_call(
        _node_body,
        grid=(N_NODES // blk,),
        in_specs=[
            pl.BlockSpec((blk, NODE_DIM), lambda i: (i, 0)),
            pl.BlockSpec((blk, NODE_DIM), lambda i: (i, 0)),
            pl.BlockSpec((_KP, blk, _F), lambda i: (0, i, 0)),
            pl.BlockSpec((_KP, blk, _F), lambda i: (0, i, 0)),
            pl.BlockSpec((blk, 1), lambda i: (i, 0)),
            pl.BlockSpec((blk, 1), lambda i: (i, 0)),
            pl.BlockSpec((NODE_DIM, HIDDEN), lambda i: (0, 0)),
            pl.BlockSpec((NODE_DIM, HIDDEN), lambda i: (0, 0)),
            pl.BlockSpec((HIDDEN, HIDDEN), lambda i: (0, 0)),
            pl.BlockSpec((1, HIDDEN), lambda i: (0, 0)),
            pl.BlockSpec((1, HIDDEN), lambda i: (0, 0)),
            pl.BlockSpec((HIDDEN, HIDDEN), lambda i: (0, 0)),
            pl.BlockSpec((1, HIDDEN), lambda i: (0, 0)),
        ],
        out_specs=pl.BlockSpec((blk, HIDDEN), lambda i: (i, 0)),
        out_shape=jax.ShapeDtypeStruct((N_NODES, HIDDEN), jnp.float32),
    )(pre_h, x, S0, S1, d0, d1, A, C, WB, bB, fn_b1.reshape(1, HIDDEN), fn_W2,
      fn_b2.reshape(1, HIDDEN))


def kernel(x, pre_h_node, edge_index, edge_attr, fe_W1, fe_b1, fe_W2, fe_b2,
           fn_W1, fn_b1, fn_W2, fn_b2):
    src = edge_index[0].astype(jnp.int32)
    dst = edge_index[1].astype(jnp.int32)
    npad = _EPAD - N_EDGES
    pad_iota = jnp.arange(npad, dtype=jnp.int32)
    src_pad = jnp.concatenate([src, pad_iota % N_NODES])
    dst_pad = jnp.concatenate([dst, N_NODES + pad_iota % (_NPAD - N_NODES)])
    ea_pad = jnp.concatenate(
        [edge_attr, jnp.zeros((npad, EDGE_DIM), jnp.float32)])
    wa = fe_W1[:NODE_DIM]
    wb = fe_W1[NODE_DIM:2 * NODE_DIM]
    wc = fe_W1[2 * NODE_DIM:]

    pqs = _compute_pq(x, wa, wb, fe_b1.reshape(1, HIDDEN))
    ps, qs = pqs[:4], pqs[4:]
    es = _compute_e(ea_pad, wc)

    S0, S1, d0, d1 = _sc_segment(ps, qs, es, src_pad, dst_pad)

    A = fn_W1[:NODE_DIM]
    B = fn_W1[NODE_DIM:NODE_DIM + HIDDEN]
    C = fn_W1[NODE_DIM + HIDDEN:]
    WB, bB = _compute_wb(fe_W2, B, fe_b2)

    return _compute_node(pre_h_node, x, S0, S1,
                         d0[:N_NODES].reshape(N_NODES, 1),
                         d1[:N_NODES].reshape(N_NODES, 1),
                         A, C, WB, bB, fn_b1, fn_W2, fn_b2)


# speedup vs baseline: 1.5085x; 1.5085x over previous
import functools

import jax
import jax.numpy as jnp
from jax import lax
from jax.experimental import pallas as pl
from jax.experimental.pallas import tpu as pltpu
from jax.experimental.pallas import tpu_sc as plsc

NODE_DIM = 256
EDGE_DIM = 16
HIDDEN = 512
N_NODES = 10000
N_EDGES = 160000

_NC = 2
_NS = 16
_NW = _NC * _NS
_L = 16

_F = 128
_KP = HIDDEN // _F
_NPAD = 10240
_RPT = _NPAD // _NS
_EPAD = 163840
_EPT = _EPAD // _NW
_G = 32
_NG = _EPT // _G
_ZR = 16


def _sc_body(p0, p1, p2, p3, q0, q1, q2, q3, e0, e1, e2, e3, src_hbm, dst_hbm,
             s0_hbm, s1_hbm, d0_hbm, d1_hbm,
             srcb, dstb,
             bufp0, bufq0, bufe0, rbuf0, bufp1, bufq1, bufe1, rbuf1,
             onesb, zbuf, dzero, shared, shared_deg,
             semp0, semq0, seme0, sems0,
             semp1, semq1, seme1, sems1, semd0, semd1):
    cid = lax.axis_index("c")
    sid = lax.axis_index("s")
    wid = sid * _NC + cid
    ebase0 = wid * _EPT
    zero16 = jnp.zeros((_L,), jnp.float32)
    one16 = jnp.full((_L,), 1.0, jnp.float32)

    pltpu.sync_copy(src_hbm.at[pl.ds(ebase0, _EPT)], srcb)
    pltpu.sync_copy(dst_hbm.at[pl.ds(ebase0, _EPT)], dstb)

    def _zb(i, _):
        for c in range(_F // _L):
            zbuf[i, pl.ds(c * _L, _L)] = zero16
        return _
    lax.fori_loop(0, _ZR, _zb, None)

    def _dz(i, _):
        dzero[pl.ds(i * _L, _L)] = zero16
        return _
    lax.fori_loop(0, _RPT // _L, _dz, None)

    def _ob(i, _):
        onesb[pl.ds(i * _L, _L)] = one16
        return _
    lax.fori_loop(0, _G // _L, _ob, None)

    rlo = sid * _RPT

    bufs = ((bufp0, bufq0, bufe0, rbuf0, semp0, semq0, seme0, sems0, semd0),
            (bufp1, bufq1, bufe1, rbuf1, semp1, semq1, seme1, sems1, semd1))

    for k, (pk, qk, ek) in enumerate(((p0, q0, e0), (p1, q1, e1),
                                      (p2, q2, e2), (p3, q3, e3))):
        def _zs(i, _):
            pltpu.sync_copy(zbuf, shared.at[pl.ds(rlo + i * _ZR, _ZR)])
            return _
        lax.fori_loop(0, _RPT // _ZR, _zs, None)
        if k == 0:
            pltpu.sync_copy(dzero, shared_deg.at[pl.ds(rlo, _RPT)])
        plsc.subcore_barrier()

        def _fire_gather(g, par, pk=pk, qk=qk, ek=ek):
            bp, bq, be = bufs[par][0], bufs[par][1], bufs[par][2]
            sp, sq, se = bufs[par][4], bufs[par][5], bufs[par][6]
            gb = g * _G
            pltpu.async_copy(pk.at[srcb.at[pl.ds(gb, _G)]], bp, sp)
            pltpu.async_copy(qk.at[dstb.at[pl.ds(gb, _G)]], bq, sq)
            pltpu.async_copy(ek.at[pl.ds(ebase0 + gb, _G)], be, se)

        def _wait_gather(par, pk=pk, qk=qk, ek=ek):
            bp, bq, be = bufs[par][0], bufs[par][1], bufs[par][2]
            sp, sq, se = bufs[par][4], bufs[par][5], bufs[par][6]
            pltpu.make_async_copy(pk.at[srcb.at[pl.ds(0, _G)]], bp, sp).wait()
            pltpu.make_async_copy(qk.at[dstb.at[pl.ds(0, _G)]], bq, sq).wait()
            pltpu.make_async_copy(ek.at[pl.ds(0, _G)], be, se).wait()

        def _wait_scat(par, k=k):
            rb, ss, sd = bufs[par][3], bufs[par][7], bufs[par][8]
            pltpu.make_async_copy(
                rb, shared.at[dstb.at[pl.ds(0, _G)]], ss).wait()
            if k == 0:
                pltpu.make_async_copy(
                    onesb, shared_deg.at[dstb.at[pl.ds(0, _G)]], sd).wait()

        def _compute_fire(g, par, k=k):
            bp, bq, be, rb = (bufs[par][0], bufs[par][1], bufs[par][2],
                              bufs[par][3])
            ss, sd = bufs[par][7], bufs[par][8]
            gb = g * _G

            def _row(j, _):
                for c in range(_F // _L):
                    col = c * _L
                    v = bp[j, pl.ds(col, _L)] + bq[j, pl.ds(col, _L)]
                    rb[j, pl.ds(col, _L)] = jnp.maximum(
                        v + be[j, pl.ds(col, _L)], 0.0)
                return _
            lax.fori_loop(0, _G, _row, None)

            pltpu.async_copy(rb, shared.at[dstb.at[pl.ds(gb, _G)]], ss,
                             add=True)
            if k == 0:
                pltpu.async_copy(onesb,
                                 shared_deg.at[dstb.at[pl.ds(gb, _G)]], sd,
                                 add=True)

        _fire_gather(0, 0)

        def _pair(i, _):
            ge = 2 * i

            _fire_gather(ge + 1, 1)
            _wait_gather(0)

            @pl.when(i > 0)
            def _():
                _wait_scat(0)
            _compute_fire(ge, 0)

            _fire_gather(ge + 2, 0)
            _wait_gather(1)

            @pl.when(i > 0)
            def _():
                _wait_scat(1)
            _compute_fire(ge + 1, 1)
            return _
        lax.fori_loop(0, (_NG - 2) // 2, _pair, None)

        _fire_gather(_NG - 1, 1)
        _wait_gather(0)
        _wait_scat(0)
        _compute_fire(_NG - 2, 0)
        _wait_gather(1)
        _wait_scat(1)
        _compute_fire(_NG - 1, 1)
        _wait_scat(0)
        _wait_scat(1)
        plsc.subcore_barrier()

        @pl.when(cid == 0)
        def _(k=k):
            pltpu.sync_copy(shared.at[pl.ds(rlo, _RPT)],
                            s0_hbm.at[k, pl.ds(rlo, _RPT)])
            if k == 0:
                pltpu.sync_copy(shared_deg.at[pl.ds(rlo, _RPT)],
                                d0_hbm.at[pl.ds(rlo, _RPT)])

        @pl.when(cid == 1)
        def _(k=k):
            pltpu.sync_copy(shared.at[pl.ds(rlo, _RPT)],
                            s1_hbm.at[k, pl.ds(rlo, _RPT)])
            if k == 0:
                pltpu.sync_copy(shared_deg.at[pl.ds(rlo, _RPT)],
                                d1_hbm.at[pl.ds(rlo, _RPT)])
        plsc.subcore_barrier()


def _sc_segment(ps, qs, es, src, dst):
    mesh = plsc.VectorSubcoreMesh(core_axis_name="c", subcore_axis_name="s")
    f = pl.kernel(
        _sc_body,
        out_type=[
            jax.ShapeDtypeStruct((_KP, _NPAD, _F), jnp.float32),
            jax.ShapeDtypeStruct((_KP, _NPAD, _F), jnp.float32),
            jax.ShapeDtypeStruct((_NPAD,), jnp.float32),
            jax.ShapeDtypeStruct((_NPAD,), jnp.float32),
        ],
        mesh=mesh,
        scratch_types=(
            [pltpu.VMEM((_EPT,), jnp.int32)] * 2
            + [pltpu.VMEM((_G, _F), jnp.float32)] * 8
            + [
                pltpu.VMEM((_G,), jnp.float32),
                pltpu.VMEM((_ZR, _F), jnp.float32),
                pltpu.VMEM((_RPT,), jnp.float32),
                pltpu.VMEM_SHARED((_NPAD, _F), jnp.float32),
                pltpu.VMEM_SHARED((_NPAD,), jnp.float32),
            ]
            + [pltpu.SemaphoreType.DMA] * 10
        ),
    )
    return f(*ps, *qs, *es, src, dst)


def _pq_body(x_ref, *refs):
    was = refs[0:4]
    wbs = refs[4:8]
    b1s = refs[8:12]
    pouts = refs[12:16]
    qouts = refs[16:20]
    xb = x_ref[:]
    for k in range(_KP):
        pouts[k][:] = (jnp.dot(xb, was[k][:], preferred_element_type=jnp.float32)
                       + b1s[k][:])
        qouts[k][:] = jnp.dot(xb, wbs[k][:], preferred_element_type=jnp.float32)


def _compute_pq(x, wa, wb, b1):
    blk = 2000
    wspec = pl.BlockSpec((NODE_DIM, _F), lambda i: (0, 0))
    bspec = pl.BlockSpec((1, _F), lambda i: (0, 0))
    ospec = pl.BlockSpec((blk, _F), lambda i: (i, 0))
    was = [wa[:, k * _F:(k + 1) * _F] for k in range(_KP)]
    wbs = [wb[:, k * _F:(k + 1) * _F] for k in range(_KP)]
    b1s = [b1[:, k * _F:(k + 1) * _F] for k in range(_KP)]
    return pl.pallas_call(
        _pq_body,
        grid=(N_NODES // blk,),
        in_specs=[pl.BlockSpec((blk, NODE_DIM), lambda i: (i, 0))]
                 + [wspec] * 8 + [bspec] * 4,
        out_specs=[ospec] * 8,
        out_shape=[jax.ShapeDtypeStruct((_NPAD, _F), jnp.float32)] * 8,
    )(x, *was, *wbs, *b1s)


def _e_body(ea_ref, *refs):
    wcs = refs[0:4]
    eouts = refs[4:8]
    eb = ea_ref[:]
    for k in range(_KP):
        eouts[k][:] = jnp.dot(eb, wcs[k][:], preferred_element_type=jnp.float32)


def _compute_e(edge_attr_pad, wc):
    blk = 8192
    return pl.pallas_call(
        _e_body,
        grid=(_EPAD // blk,),
        in_specs=[pl.BlockSpec((blk, EDGE_DIM), lambda i: (i, 0))]
                 + [pl.BlockSpec((EDGE_DIM, _F), lambda i: (0, 0))] * 4,
        out_specs=[pl.BlockSpec((blk, _F), lambda i: (i, 0))] * 4,
        out_shape=[jax.ShapeDtypeStruct((_EPAD, _F), jnp.float32)] * 4,
    )(edge_attr_pad, *[wc[:, k * _F:(k + 1) * _F] for k in range(_KP)])


def _wb_body(w2_ref, b_ref, b2_ref, wb_ref, bb_ref):
    wb_ref[:] = jnp.dot(w2_ref[:], b_ref[:], preferred_element_type=jnp.float32)
    bb_ref[:] = jnp.dot(b2_ref[:], b_ref[:], preferred_element_type=jnp.float32)


def _compute_wb(fe_W2, B, fe_b2):
    return pl.pallas_call(
        _wb_body,
        out_shape=[
            jax.ShapeDtypeStruct((HIDDEN, HIDDEN), jnp.float32),
            jax.ShapeDtypeStruct((1, HIDDEN), jnp.float32),
        ],
    )(fe_W2, B, fe_b2.reshape(1, HIDDEN))


def _node_body(ph_ref, x_ref, s0_ref, s1_ref, d0_ref, d1_ref, a_ref, c_ref,
               wb_ref, bb_ref, b1_ref, w2_ref, b2_ref, o_ref):
    z = jnp.dot(ph_ref[:], a_ref[:], preferred_element_type=jnp.float32)
    z += jnp.dot(x_ref[:], c_ref[:], preferred_element_type=jnp.float32)
    for k in range(_KP):
        z += jnp.dot(s0_ref[k] + s1_ref[k], wb_ref[k * _F:(k + 1) * _F, :],
                     preferred_element_type=jnp.float32)
    z += (d0_ref[:] + d1_ref[:]) * bb_ref[:]
    z += b1_ref[:]
    z = jax.nn.relu(z)
    o_ref[:] = jnp.dot(z, w2_ref[:], preferred_element_type=jnp.float32) + b2_ref[:]


def _compute_node(pre_h, x, S0, S1, d0, d1, A, C, WB, bB, fn_b1, fn_W2, fn_b2):
    blk = 2000
    return pl.pallas_call(
        _node_body,
        grid=(N_NODES // blk,),
        in_specs=[
            pl.BlockSpec((blk, NODE_DIM), lambda i: (i, 0)),
            pl.BlockSpec((blk, NODE_DIM), lambda i: (i, 0)),
            pl.BlockSpec((_KP, blk, _F), lambda i: (0, i, 0)),
            pl.BlockSpec((_KP, blk, _F), lambda i: (0, i, 0)),
            pl.BlockSpec((blk, 1), lambda i: (i, 0)),
            pl.BlockSpec((blk, 1), lambda i: (i, 0)),
            pl.BlockSpec((NODE_DIM, HIDDEN), lambda i: (0, 0)),
            pl.BlockSpec((NODE_DIM, HIDDEN), lambda i: (0, 0)),
            pl.BlockSpec((HIDDEN, HIDDEN), lambda i: (0, 0)),
            pl.BlockSpec((1, HIDDEN), lambda i: (0, 0)),
            pl.BlockSpec((1, HIDDEN), lambda i: (0, 0)),
            pl.BlockSpec((HIDDEN, HIDDEN), lambda i: (0, 0)),
            pl.BlockSpec((1, HIDDEN), lambda i: (0, 0)),
        ],
        out_specs=pl.BlockSpec((blk, HIDDEN), lambda i: (i, 0)),
        out_shape=jax.ShapeDtypeStruct((N_NODES, HIDDEN), jnp.float32),
    )(pre_h, x, S0, S1, d0, d1, A, C, WB, bB, fn_b1.reshape(1, HIDDEN), fn_W2,
      fn_b2.reshape(1, HIDDEN))


def kernel(x, pre_h_node, edge_index, edge_attr, fe_W1, fe_b1, fe_W2, fe_b2,
           fn_W1, fn_b1, fn_W2, fn_b2):
    src = edge_index[0].astype(jnp.int32)
    dst = edge_index[1].astype(jnp.int32)
    npad = _EPAD - N_EDGES
    pad_iota = jnp.arange(npad, dtype=jnp.int32)
    src_pad = jnp.concatenate([src, pad_iota % N_NODES])
    dst_pad = jnp.concatenate([dst, N_NODES + pad_iota % (_NPAD - N_NODES)])
    ea_pad = jnp.concatenate(
        [edge_attr, jnp.zeros((npad, EDGE_DIM), jnp.float32)])
    wa = fe_W1[:NODE_DIM]
    wb = fe_W1[NODE_DIM:2 * NODE_DIM]
    wc = fe_W1[2 * NODE_DIM:]

    pqs = _compute_pq(x, wa, wb, fe_b1.reshape(1, HIDDEN))
    ps, qs = pqs[:4], pqs[4:]
    es = _compute_e(ea_pad, wc)

    S0, S1, d0, d1 = _sc_segment(ps, qs, es, src_pad, dst_pad)

    A = fn_W1[:NODE_DIM]
    B = fn_W1[NODE_DIM:NODE_DIM + HIDDEN]
    C = fn_W1[NODE_DIM + HIDDEN:]
    WB, bB = _compute_wb(fe_W2, B, fe_b2)

    return _compute_node(pre_h_node, x, S0, S1,
                         d0[:N_NODES].reshape(N_NODES, 1),
                         d1[:N_NODES].reshape(N_NODES, 1),
                         A, C, WB, bB, fn_b1, fn_W2, fn_b2)
